# baseline (speedup 1.0000x reference)
import functools

import jax
import jax.numpy as jnp
from jax import lax
from jax.experimental import pallas as pl
from jax.experimental.pallas import tpu as pltpu
from jax.experimental.pallas import tpu_sc as plsc

_INFO = plsc.get_sparse_core_info()
_NC = _INFO.num_cores
_NS = _INFO.num_subcores
_NW = _NC * _NS


def _transpose16(src, dst, nrows, a_len):
    lanes = lax.iota(jnp.int32, 16)
    rots = [lax.rem(lanes + r, 16) for r in range(16)]
    assert nrows % 16 == 0 and a_len % 16 == 0

    def grp(g, carry):
        cols = g * 16 + lanes
        for k0 in range(0, a_len // 16, 2):
            ks = [k0] if k0 + 1 >= a_len // 16 else [k0, k0 + 1]
            batch = [(rots[r] + (16 * k)) for k in ks for r in range(16)]
            vs = [plsc.load_gather(src, [rows, cols]) for rows in batch]
            for rows, v in zip(batch, vs):
                plsc.store_scatter(dst, [cols, rows], v)
        return carry

    lax.fori_loop(0, nrows // 16, grp, 0)


def _make_fused(n_j, n_i, dim, vocab):
    assert n_i % 128 == 0 and dim == 64 and n_i // 128 == _NW
    n_tcol = vocab // 128
    tail = vocab - n_tcol * 128
    n_jt = n_j // 8
    j_tail = n_j - n_jt * 8
    mesh = plsc.VectorSubcoreMesh(core_axis_name="c", subcore_axis_name="s")

    @functools.partial(
        pl.kernel,
        mesh=mesh,
        out_type=(
            jax.ShapeDtypeStruct((n_j, dim, n_i), jnp.float32),
            jax.ShapeDtypeStruct((_NC * vocab, 128), jnp.float32),
        ),
        scratch_types=[
            pltpu.VMEM((2, 256, 128), jnp.float32),
            pltpu.VMEM((2, 2, 64, 128), jnp.float32),
            pltpu.VMEM((8, 128), jnp.int32),
            pltpu.VMEM((j_tail, 128), jnp.int32),
            pltpu.VMEM((256,), jnp.int32),
            pltpu.VMEM((256,), jnp.int32),
            pltpu.VMEM((tail, dim), jnp.float32),
            pltpu.VMEM((544,), jnp.float32),
            pltpu.SemaphoreType.DMA,
            pltpu.SemaphoreType.DMA,
            pltpu.SemaphoreType.DMA,
            pltpu.SemaphoreType.DMA,
        ],
        compiler_params=pltpu.CompilerParams(needs_layout_passes=False),
    )
    def fused(tt_hbm, wt_hbm, wtail_hbm, ttail_hbm, o3_hbm, w2_hbm,
              big, small, tilebuf, jtailbuf, idx0, idx1, vtail, stage,
              g0, g1, w0, w1):
        cid = lax.axis_index("c")
        sid = lax.axis_index("s")
        wid = sid * _NC + cid
        gsem = (g0, g1)
        wsem = (w0, w1)
        idxs = (idx0, idx1)
        w2_base = cid * vocab

        def p1_write(b, buf):
            pltpu.async_copy(big.at[buf, pl.ds(0, 128)],
                             w2_hbm.at[pl.ds(pl.multiple_of(w2_base + b * 128, 8), 128)],
                             wsem[buf])

        def p1_write_wait(b, buf):
            pltpu.make_async_copy(big.at[buf, pl.ds(0, 128)],
                                  w2_hbm.at[pl.ds(pl.multiple_of(w2_base + b * 128, 8), 128)],
                                  wsem[buf]).wait()

        def p1_read(b, buf):
            pltpu.async_copy(wt_hbm.at[:, pl.ds(b * 128, 128)],
                             small.at[buf, 0], gsem[buf])

        def p1_read_wait(b, buf):
            pltpu.make_async_copy(wt_hbm.at[:, pl.ds(b * 128, 128)],
                                  small.at[buf, 0], gsem[buf]).wait()

        def p1_step(t, buf):
            b = sid + t * _NS

            @pl.when(b < n_tcol)
            def _():
                @pl.when(b + _NS < n_tcol)
                def _():
                    p1_read(b + _NS, 1 - buf)

                p1_read_wait(b, buf)

                @pl.when(t >= 2)
                def _():
                    p1_write_wait(sid + (t - 2) * _NS, buf)

                _transpose16(small.at[buf, 0], big.at[buf, pl.ds(0, 128)], 128, 64)
                p1_write(b, buf)

        def p1_outer(g, carry):
            for half in range(2):
                p1_step(g * 2 + half, half)
            return carry

        n_p1 = (n_tcol + _NS - 1) // _NS

        @pl.when(sid < n_tcol)
        def _():
            p1_read(sid, 0)

        lax.fori_loop(0, (n_p1 + 1) // 2, p1_outer, 0)

        sid_full = (n_tcol - 1) % _NS

        @pl.when(sid <= sid_full)
        def _():
            for t in (n_p1 - 2, n_p1 - 1):
                p1_write_wait(sid + t * _NS, t % 2)

        @pl.when(sid > sid_full)
        def _():
            for t in (n_p1 - 3, n_p1 - 2):
                p1_write_wait(sid + t * _NS, t % 2)

        @pl.when(sid == _NS - 1)
        def _():
            pltpu.sync_copy(wtail_hbm, vtail)
            for r in range(tail):
                for k in range(dim // 16):
                    big[0, r, pl.ds(k * 16, 16)] = vtail[r, pl.ds(k * 16, 16)]
            pltpu.sync_copy(
                big.at[0, pl.ds(0, tail)],
                w2_hbm.at[pl.ds(pl.multiple_of(w2_base + n_tcol * 128, 8), tail)])

        plsc.subcore_barrier()

        col0 = wid * 128
        base_splat = jnp.full((16,), w2_base, jnp.int32)

        def idx_stage(t, buf):
            j0 = 2 * t
            @pl.when(jnp.logical_and(lax.rem(j0, 8) == 0, j0 < n_jt * 8))
            def _():
                pltpu.sync_copy(
                    tt_hbm.at[pl.ds(pl.multiple_of(j0, 8), 8),
                              pl.ds(col0, 128)], tilebuf)

            @pl.when(j0 == n_jt * 8)
            def _():
                pltpu.sync_copy(ttail_hbm.at[:, pl.ds(col0, 128)], jtailbuf)

            jrow = lax.rem(j0, 8)
            for h in range(2):
                for k in range(8):
                    sl = pl.ds(k * 16, 16)
                    dsl = pl.ds(h * 128 + k * 16, 16)

                    @pl.when(j0 < n_jt * 8)
                    def _():
                        idxs[buf][dsl] = tilebuf[jrow + h, sl] + base_splat

                    @pl.when(j0 >= n_jt * 8)
                    def _():
                        idxs[buf][dsl] = jtailbuf[h, sl] + base_splat

        def gather_start(buf):
            pltpu.async_copy(w2_hbm.at[idxs[buf]], big.at[buf], gsem[buf])

        def gather_wait(buf):
            pltpu.make_async_copy(w2_hbm.at[idxs[buf]], big.at[buf],
                                  gsem[buf]).wait()

        def out_write(t, buf, h):
            pltpu.async_copy(small.at[buf, h],
                             o3_hbm.at[2 * t + h, :, pl.ds(col0, 128)],
                             wsem[buf])

        def out_write_wait(t, buf, h):
            pltpu.make_async_copy(small.at[buf, h],
                                  o3_hbm.at[2 * t + h, :, pl.ds(col0, 128)],
                                  wsem[buf]).wait()

        n_super = n_j // 2
        idx_stage(0, 0)
        gather_start(0)

        def p2_step(t, buf):
            nbuf = 1 - buf

            @pl.when(t + 1 < n_super)
            def _():
                idx_stage(t + 1, nbuf)

                @pl.when(t >= 1)
                def _():
                    out_write_wait(t - 1, nbuf, 0)
                    out_write_wait(t - 1, nbuf, 1)

                gather_start(nbuf)

            gather_wait(buf)
            for h in range(2):
                _transpose16(big.at[buf, pl.ds(h * 128, 128)],
                             small.at[buf, h], 64, 128)
                out_write(t, buf, h)

        def p2_outer(g, carry):
            for half in range(2):
                p2_step(g * 2 + half, half)
            return carry

        lax.fori_loop(0, n_super // 2, p2_outer, 0)
        p2_step(n_super - 1, (n_super - 1) % 2)
        for h in range(2):
            out_write_wait(n_super - 2, (n_super - 2) % 2, h)
            out_write_wait(n_super - 1, (n_super - 1) % 2, h)

    return fused


def kernel(token_ids, weights):
    vocab, dim = weights.shape
    n_i, n_j = token_ids.shape
    tt = token_ids.T.astype(jnp.int32)
    wt = weights.T
    wtail = weights[vocab - (vocab % 128):]
    ttail = tt[(n_j // 8) * 8:]
    o3, _ = _make_fused(n_j, n_i, dim, vocab)(tt, wt, wtail, ttail)
    return jnp.transpose(o3, (2, 0, 1))


# speedup vs baseline: 1.0466x; 1.0466x over previous
import functools

import jax
import jax.numpy as jnp
from jax import lax
from jax.experimental import pallas as pl
from jax.experimental.pallas import tpu as pltpu
from jax.experimental.pallas import tpu_sc as plsc

_INFO = plsc.get_sparse_core_info()
_NC = _INFO.num_cores
_NS = _INFO.num_subcores
_NW = _NC * _NS


def _transpose16(src, dst, nrows, a_len):
    lanes = lax.iota(jnp.int32, 16)
    rots = [lax.rem(lanes + r, 16) for r in range(16)]
    assert nrows % 16 == 0 and a_len % 16 == 0

    def grp(g, carry):
        cols = g * 16 + lanes
        for k in range(a_len // 16):
            rows_k = [rots[r] + (16 * k) for r in range(16)]
            vs = [plsc.load_gather(src, [rows_k[r], cols])
                  for r in range(16)]
            for r in range(16):
                plsc.store_scatter(dst, [cols, rows_k[r]], vs[r])
        return carry

    lax.fori_loop(0, nrows // 16, grp, 0)


def _make_fused(n_j, n_i, dim, vocab):
    assert n_i % 128 == 0 and dim == 64 and n_i // 128 == _NW
    n_tcol = vocab // 128
    tail = vocab - n_tcol * 128
    n_jt = n_j // 8
    j_tail = n_j - n_jt * 8
    mesh = plsc.VectorSubcoreMesh(core_axis_name="c", subcore_axis_name="s")

    @functools.partial(
        pl.kernel,
        mesh=mesh,
        out_type=(
            jax.ShapeDtypeStruct((n_j, dim, n_i), jnp.float32),
            jax.ShapeDtypeStruct((_NC * vocab, 128), jnp.float32),
        ),
        scratch_types=[
            pltpu.VMEM((2, 256, 128), jnp.float32),
            pltpu.VMEM((2, 2, 64, 128), jnp.float32),
            pltpu.VMEM((8, 128), jnp.int32),
            pltpu.VMEM((j_tail, 128), jnp.int32),
            pltpu.VMEM((256,), jnp.int32),
            pltpu.VMEM((256,), jnp.int32),
            pltpu.VMEM((tail, dim), jnp.float32),
            pltpu.VMEM((544,), jnp.float32),
            pltpu.SemaphoreType.DMA,
            pltpu.SemaphoreType.DMA,
            pltpu.SemaphoreType.DMA,
            pltpu.SemaphoreType.DMA,
        ],
        compiler_params=pltpu.CompilerParams(needs_layout_passes=False),
    )
    def fused(tt_hbm, wt_hbm, wtail_hbm, ttail_hbm, o3_hbm, w2_hbm,
              big, small, tilebuf, jtailbuf, idx0, idx1, vtail, stage,
              g0, g1, w0, w1):
        cid = lax.axis_index("c")
        sid = lax.axis_index("s")
        wid = sid * _NC + cid
        gsem = (g0, g1)
        wsem = (w0, w1)
        idxs = (idx0, idx1)
        w2_base = cid * vocab

        def p1_write(b, buf):
            pltpu.async_copy(big.at[buf, pl.ds(0, 128)],
                             w2_hbm.at[pl.ds(pl.multiple_of(w2_base + b * 128, 8), 128)],
                             wsem[buf])

        def p1_write_wait(b, buf):
            pltpu.make_async_copy(big.at[buf, pl.ds(0, 128)],
                                  w2_hbm.at[pl.ds(pl.multiple_of(w2_base + b * 128, 8), 128)],
                                  wsem[buf]).wait()

        def p1_read(b, buf):
            pltpu.async_copy(wt_hbm.at[:, pl.ds(b * 128, 128)],
                             small.at[buf, 0], gsem[buf])

        def p1_read_wait(b, buf):
            pltpu.make_async_copy(wt_hbm.at[:, pl.ds(b * 128, 128)],
                                  small.at[buf, 0], gsem[buf]).wait()

        def p1_step(t, buf):
            b = sid + t * _NS

            @pl.when(b < n_tcol)
            def _():
                @pl.when(b + _NS < n_tcol)
                def _():
                    p1_read(b + _NS, 1 - buf)

                p1_read_wait(b, buf)

                @pl.when(t >= 2)
                def _():
                    p1_write_wait(sid + (t - 2) * _NS, buf)

                _transpose16(small.at[buf, 0], big.at[buf, pl.ds(0, 128)], 128, 64)
                p1_write(b, buf)

        def p1_outer(g, carry):
            for half in range(2):
                p1_step(g * 2 + half, half)
            return carry

        n_p1 = (n_tcol + _NS - 1) // _NS

        @pl.when(sid < n_tcol)
        def _():
            p1_read(sid, 0)

        lax.fori_loop(0, (n_p1 + 1) // 2, p1_outer, 0)

        sid_full = (n_tcol - 1) % _NS

        @pl.when(sid <= sid_full)
        def _():
            for t in (n_p1 - 2, n_p1 - 1):
                p1_write_wait(sid + t * _NS, t % 2)

        @pl.when(sid > sid_full)
        def _():
            for t in (n_p1 - 3, n_p1 - 2):
                p1_write_wait(sid + t * _NS, t % 2)

        @pl.when(sid == _NS - 1)
        def _():
            pltpu.sync_copy(wtail_hbm, vtail)
            for r in range(tail):
                for k in range(dim // 16):
                    big[0, r, pl.ds(k * 16, 16)] = vtail[r, pl.ds(k * 16, 16)]
            pltpu.sync_copy(
                big.at[0, pl.ds(0, tail)],
                w2_hbm.at[pl.ds(pl.multiple_of(w2_base + n_tcol * 128, 8), tail)])

        plsc.subcore_barrier()

        col0 = wid * 128
        base_splat = jnp.full((16,), w2_base, jnp.int32)

        def idx_stage(t, buf):
            j0 = 2 * t
            @pl.when(jnp.logical_and(lax.rem(j0, 8) == 0, j0 < n_jt * 8))
            def _():
                pltpu.sync_copy(
                    tt_hbm.at[pl.ds(pl.multiple_of(j0, 8), 8),
                              pl.ds(col0, 128)], tilebuf)

            @pl.when(j0 == n_jt * 8)
            def _():
                pltpu.sync_copy(ttail_hbm.at[:, pl.ds(col0, 128)], jtailbuf)

            jrow = lax.rem(j0, 8)
            for h in range(2):
                for k in range(8):
                    sl = pl.ds(k * 16, 16)
                    dsl = pl.ds(h * 128 + k * 16, 16)

                    @pl.when(j0 < n_jt * 8)
                    def _():
                        idxs[buf][dsl] = tilebuf[jrow + h, sl] + base_splat

                    @pl.when(j0 >= n_jt * 8)
                    def _():
                        idxs[buf][dsl] = jtailbuf[h, sl] + base_splat

        def gather_start(buf):
            pltpu.async_copy(w2_hbm.at[idxs[buf]], big.at[buf], gsem[buf])

        def gather_wait(buf):
            pltpu.make_async_copy(w2_hbm.at[idxs[buf]], big.at[buf],
                                  gsem[buf]).wait()

        def out_write(t, buf, h):
            pltpu.async_copy(small.at[buf, h],
                             o3_hbm.at[2 * t + h, :, pl.ds(col0, 128)],
                             wsem[buf])

        def out_write_wait(t, buf, h):
            pltpu.make_async_copy(small.at[buf, h],
                                  o3_hbm.at[2 * t + h, :, pl.ds(col0, 128)],
                                  wsem[buf]).wait()

        n_super = n_j // 2
        idx_stage(0, 0)
        gather_start(0)

        def p2_step(t, buf):
            nbuf = 1 - buf

            @pl.when(t + 1 < n_super)
            def _():
                idx_stage(t + 1, nbuf)

                @pl.when(t >= 1)
                def _():
                    out_write_wait(t - 1, nbuf, 0)
                    out_write_wait(t - 1, nbuf, 1)

                gather_start(nbuf)

            gather_wait(buf)
            for h in range(2):
                _transpose16(big.at[buf, pl.ds(h * 128, 128)],
                             small.at[buf, h], 64, 128)
                out_write(t, buf, h)

        def p2_outer(g, carry):
            for half in range(2):
                p2_step(g * 2 + half, half)
            return carry

        lax.fori_loop(0, n_super // 2, p2_outer, 0)
        p2_step(n_super - 1, (n_super - 1) % 2)
        for h in range(2):
            out_write_wait(n_super - 2, (n_super - 2) % 2, h)
            out_write_wait(n_super - 1, (n_super - 1) % 2, h)

    return fused


def kernel(token_ids, weights):
    vocab, dim = weights.shape
    n_i, n_j = token_ids.shape
    tt = token_ids.T.astype(jnp.int32)
    wt = weights.T
    wtail = weights[vocab - (vocab % 128):]
    ttail = tt[(n_j // 8) * 8:]
    o3, _ = _make_fused(n_j, n_i, dim, vocab)(tt, wt, wtail, ttail)
    return jnp.transpose(o3, (2, 0, 1))
